# baseline (speedup 1.0000x reference)
import jax
import jax.numpy as jnp
from jax.experimental import pallas as pl
from jax.experimental.pallas import tpu as pltpu

NQ = 1024
D = 64
NKV = 100000
BK = 4096
NPAD = -(-NKV // BK) * BK - NKV
LOG2E = 1.4426950408889634


def _attn_kernel(lat_ref, kt_ref, vt_ref, o_ref, q_ref, vaug_ref, acc_ref):
    i = pl.program_id(1)
    nb = pl.num_programs(1)

    @pl.when(i == 0)
    def _init():
        lat = lat_ref[:]
        n = jnp.sqrt(jnp.sum(lat * lat, axis=1, keepdims=True))
        q_ref[:] = (lat / jnp.maximum(n, 1e-12)).astype(jnp.bfloat16)
        acc_ref[:] = jnp.zeros_like(acc_ref)
        vaug_ref[D:, :] = jnp.ones((8, BK), jnp.bfloat16)

    kt = kt_ref[:]
    col = i * BK + jax.lax.broadcasted_iota(jnp.int32, (1, BK), 1)
    valid = col < NKV
    sq = jax.lax.dot_general(
        jnp.ones((1, D), jnp.float32), kt * kt,
        (((1,), (0,)), ((), ())), preferred_element_type=jnp.float32)
    inv = jax.lax.rsqrt(jnp.maximum(sq, 1e-24)) * LOG2E
    ktn = jnp.where(valid, kt * inv, 0.0).astype(jnp.bfloat16)
    raw = jax.lax.dot_general(
        q_ref[:], ktn, (((1,), (0,)), ((), ())),
        preferred_element_type=jnp.float32)
    eb = jnp.exp2(raw).astype(jnp.bfloat16)
    vaug_ref[:D, :] = jnp.where(valid, vt_ref[:], 0.0).astype(jnp.bfloat16)
    acc_ref[:] += jax.lax.dot_general(
        eb, vaug_ref[:], (((1,), (1,)), ((), ())),
        preferred_element_type=jnp.float32)

    @pl.when(i == nb - 1)
    def _finish():
        acc = acc_ref[:]
        o_ref[:] = acc[:, :D] / (acc[:, D:D + 1] - float(NPAD))


def kernel(latent, keys, vals):
    nb = pl.cdiv(NKV, BK)
    ktall = keys.T
    vtall = vals.T
    return pl.pallas_call(
        _attn_kernel,
        grid=(2, nb),
        in_specs=[
            pl.BlockSpec((NQ // 2, D), lambda qi, i: (qi, 0)),
            pl.BlockSpec((D, BK), lambda qi, i: (0, i)),
            pl.BlockSpec((D, BK), lambda qi, i: (0, i)),
        ],
        out_specs=pl.BlockSpec((NQ // 2, D), lambda qi, i: (qi, 0)),
        out_shape=jax.ShapeDtypeStruct((NQ, D), jnp.float32),
        scratch_shapes=[
            pltpu.VMEM((NQ // 2, D), jnp.bfloat16),
            pltpu.VMEM((D + 8, BK), jnp.bfloat16),
            pltpu.VMEM((NQ // 2, D + 8), jnp.float32),
        ],
        compiler_params=pltpu.CompilerParams(
            dimension_semantics=("parallel", "arbitrary")),
    )(latent, ktall, vtall)


# speedup vs baseline: 1.1247x; 1.1247x over previous
import jax
import jax.numpy as jnp
from jax.experimental import pallas as pl
from jax.experimental.pallas import tpu as pltpu

NQ = 1024
D = 64
NKV = 100000
BK = 4096
NPAD = -(-NKV // BK) * BK - NKV
LOG2E = 1.4426950408889634


def _attn_kernel(latT_ref, kt_ref, vt_ref, o_ref, qT_ref, vaug_ref, acc_ref):
    i = pl.program_id(0)
    nb = pl.num_programs(0)

    @pl.when(i == 0)
    def _init():
        lt = latT_ref[:]
        nsq = jnp.sum(lt * lt, axis=0, keepdims=True)
        qT_ref[:] = (lt * jax.lax.rsqrt(jnp.maximum(nsq, 1e-24))).astype(
            jnp.bfloat16)
        acc_ref[:] = jnp.zeros_like(acc_ref)
        vaug_ref[D:, :] = jnp.ones((8, BK), jnp.bfloat16)

    kt = kt_ref[:]
    col = i * BK + jax.lax.broadcasted_iota(jnp.int32, (1, BK), 1)
    valid = col < NKV
    sq = jnp.sum(kt * kt, axis=0, keepdims=True)
    inv = jax.lax.rsqrt(jnp.maximum(sq, 1e-24)) * LOG2E
    ktn = jnp.where(valid, kt * inv, 0.0).astype(jnp.bfloat16)
    raw = jax.lax.dot_general(
        qT_ref[:], ktn, (((0,), (0,)), ((), ())),
        preferred_element_type=jnp.float32)
    eb = jnp.exp2(raw).astype(jnp.bfloat16)
    vaug_ref[:D, :] = jnp.where(valid, vt_ref[:], 0.0).astype(jnp.bfloat16)
    acc_ref[:] += jax.lax.dot_general(
        eb, vaug_ref[:], (((1,), (1,)), ((), ())),
        preferred_element_type=jnp.float32)

    @pl.when(i == nb - 1)
    def _finish():
        acc = acc_ref[:]
        o_ref[:] = acc[:, :D] / (acc[:, D:D + 1] - float(NPAD))


def kernel(latent, keys, vals):
    nb = pl.cdiv(NKV, BK)
    latT = latent.T
    ktall = keys.T
    vtall = vals.T
    return pl.pallas_call(
        _attn_kernel,
        grid=(nb,),
        in_specs=[
            pl.BlockSpec((D, NQ), lambda i: (0, 0)),
            pl.BlockSpec((D, BK), lambda i: (0, i)),
            pl.BlockSpec((D, BK), lambda i: (0, i)),
        ],
        out_specs=pl.BlockSpec((NQ, D), lambda i: (0, 0)),
        out_shape=jax.ShapeDtypeStruct((NQ, D), jnp.float32),
        scratch_shapes=[
            pltpu.VMEM((D, NQ), jnp.bfloat16),
            pltpu.VMEM((D + 8, BK), jnp.bfloat16),
            pltpu.VMEM((NQ, D + 8), jnp.float32),
        ],
    )(latT, ktall, vtall)


# speedup vs baseline: 1.1916x; 1.0595x over previous
import jax
import jax.numpy as jnp
from jax.experimental import pallas as pl
from jax.experimental.pallas import tpu as pltpu

NQ = 1024
D = 64
NKV = 100000
BK = 7168
NPAD = -(-NKV // BK) * BK - NKV
LOG2E = 1.4426950408889634


def _attn_kernel(latT_ref, kt_ref, vt_ref, o_ref, qT_ref, vaug_ref, acc_ref):
    i = pl.program_id(0)
    nb = pl.num_programs(0)

    @pl.when(i == 0)
    def _init():
        lt = latT_ref[:]
        nsq = jnp.sum(lt * lt, axis=0, keepdims=True)
        qT_ref[:] = (lt * jax.lax.rsqrt(jnp.maximum(nsq, 1e-24))).astype(
            jnp.bfloat16)
        acc_ref[:] = jnp.zeros_like(acc_ref)
        vaug_ref[D:, :] = jnp.ones((8, BK), jnp.bfloat16)

    kt = kt_ref[:]
    col = i * BK + jax.lax.broadcasted_iota(jnp.int32, (1, BK), 1)
    valid = col < NKV
    sq = jnp.sum(kt * kt, axis=0, keepdims=True)
    inv = jax.lax.rsqrt(jnp.maximum(sq, 1e-24)) * LOG2E
    ktn = jnp.where(valid, kt * inv, 0.0).astype(jnp.bfloat16)
    raw = jax.lax.dot_general(
        qT_ref[:], ktn, (((0,), (0,)), ((), ())),
        preferred_element_type=jnp.float32)
    eb = jnp.exp2(raw).astype(jnp.bfloat16)
    vaug_ref[:D, :] = jnp.where(valid, vt_ref[:], 0.0).astype(jnp.bfloat16)
    acc_ref[:] += jax.lax.dot_general(
        eb, vaug_ref[:], (((1,), (1,)), ((), ())),
        preferred_element_type=jnp.float32)

    @pl.when(i == nb - 1)
    def _finish():
        acc = acc_ref[:]
        o_ref[:] = acc[:, :D] / (acc[:, D:D + 1] - float(NPAD))


def kernel(latent, keys, vals):
    nb = pl.cdiv(NKV, BK)
    latT = latent.T
    ktall = keys.T
    vtall = vals.T
    return pl.pallas_call(
        _attn_kernel,
        grid=(nb,),
        in_specs=[
            pl.BlockSpec((D, NQ), lambda i: (0, 0)),
            pl.BlockSpec((D, BK), lambda i: (0, i)),
            pl.BlockSpec((D, BK), lambda i: (0, i)),
        ],
        out_specs=pl.BlockSpec((NQ, D), lambda i: (0, 0)),
        out_shape=jax.ShapeDtypeStruct((NQ, D), jnp.float32),
        scratch_shapes=[
            pltpu.VMEM((D, NQ), jnp.bfloat16),
            pltpu.VMEM((D + 8, BK), jnp.bfloat16),
            pltpu.VMEM((NQ, D + 8), jnp.float32),
        ],
    )(latT, ktall, vtall)
